# Initial kernel scaffold; baseline (speedup 1.0000x reference)
import jax
import jax.numpy as jnp
from jax.experimental import pallas as pl


def kernel(x, path_lengths, params):
    raise NotImplementedError("write your pallas kernel here")


# speedup vs baseline: 1.8448x; 1.8448x over previous
import functools

import jax
import jax.numpy as jnp
from jax.experimental import pallas as pl
from jax.experimental.pallas import tpu as pltpu

_IN = 1024
_OUT = 2048
_MAXL = 8
_T = 256
_NTOK = 8192
_NB = _NTOK // _T + _MAXL
_DEPTH = (3, 3, 4, 4, 5, 5, 5, 5)
_MAXD = 5

_SLOT = []
_slot_base = 0
for _e in range(_MAXL):
    _SLOT.append([_slot_base + _j for _j in range(_DEPTH[_e] - 1)])
    _slot_base += _DEPTH[_e] - 1
_NSLOTS = _slot_base

_RSEL_ROWS = []
for _e in range(_MAXL):
    _s = _SLOT[_e]
    _row = [_s[0], _s[0]] + [_s[min(_j, len(_s) - 1)] for _j in range(1, _MAXD - 1)]
    _RSEL_ROWS.append(_row)
_LAST_SLOT = [_SLOT[_e][-1] for _e in range(_MAXL)]


def _mlp_body(sel_ref, rsel_ref, islayer_ref, islast_ref,
              x_ref, w0_ref, b0_ref, wr_ref, br_ref,
              o_ref, h_ref):
    b = pl.program_id(0)
    l = pl.program_id(1)
    do = islayer_ref[b, l] == 1
    last = islast_ref[b, l] == 1

    @pl.when(do & (l == 0))
    def _first():
        acc = jnp.dot(x_ref[...], w0_ref[0],
                      preferred_element_type=jnp.float32) + b0_ref[0]
        h_ref[...] = jnp.maximum(acc, 0.0)

    @pl.when(do & (l > 0) & jnp.logical_not(last))
    def _mid():
        acc = jnp.dot(h_ref[...], wr_ref[0],
                      preferred_element_type=jnp.float32) + br_ref[0]
        h_ref[...] = jnp.maximum(acc, 0.0)

    @pl.when(do & last)
    def _final():
        o_ref[...] = jnp.dot(h_ref[...], wr_ref[0],
                             preferred_element_type=jnp.float32) + br_ref[0]


@functools.partial(jax.jit, static_argnames=())
def _expert_mlp(x_sorted, sel, rsel, is_layer, is_last,
                w0_bank, b0_bank, wr_bank, br_bank):
    grid_spec = pltpu.PrefetchScalarGridSpec(
        num_scalar_prefetch=4,
        grid=(_NB, _MAXD),
        in_specs=[
            pl.BlockSpec((_T, _IN), lambda b, l, *p: (b, 0)),
            pl.BlockSpec((1, _IN, _OUT), lambda b, l, sel, rsel, *p: (sel[b], 0, 0)),
            pl.BlockSpec((1, 1, _OUT), lambda b, l, sel, rsel, *p: (sel[b], 0, 0)),
            pl.BlockSpec((1, _OUT, _OUT), lambda b, l, sel, rsel, *p: (rsel[b, l], 0, 0)),
            pl.BlockSpec((1, 1, _OUT), lambda b, l, sel, rsel, *p: (rsel[b, l], 0, 0)),
        ],
        out_specs=pl.BlockSpec((_T, _OUT), lambda b, l, *p: (b, 0)),
        scratch_shapes=[pltpu.VMEM((_T, _OUT), jnp.float32)],
    )
    return pl.pallas_call(
        _mlp_body,
        grid_spec=grid_spec,
        out_shape=jax.ShapeDtypeStruct((_NB * _T, _OUT), jnp.float32),
        compiler_params=pltpu.CompilerParams(
            dimension_semantics=("arbitrary", "arbitrary")),
    )(sel, rsel, is_layer, is_last,
      x_sorted, w0_bank, b0_bank, wr_bank, br_bank)


def _pack_weights(params):
    w0s, b0s, wrs, brs = [], [], [], []
    for e in range(_MAXL):
        layers = params[e]
        W0, B0 = layers[0]
        d0 = W0.shape[0]
        w0s.append(jnp.pad(W0.T, ((0, 0), (0, _OUT - d0))))
        b0s.append(jnp.pad(B0, (0, _OUT - d0)))
        for j in range(1, _DEPTH[e]):
            W, B = layers[j]
            dout, din = W.shape
            wrs.append(jnp.pad(W.T, ((0, _OUT - din), (0, _OUT - dout))))
            brs.append(jnp.pad(B, (0, _OUT - dout)))
    return (jnp.stack(w0s), jnp.stack(b0s)[:, None, :],
            jnp.stack(wrs), jnp.stack(brs)[:, None, :])


def kernel(x, path_lengths, params):
    b, n, d = x.shape
    xf = x.reshape(b * n, d)
    plf = jnp.clip(path_lengths.reshape(b * n), 0, _MAXL - 1)

    onehot = (plf[:, None] == jnp.arange(_MAXL, dtype=jnp.int32)[None, :])
    oh32 = onehot.astype(jnp.int32)
    counts = jnp.sum(oh32, axis=0)
    padded = ((counts + _T - 1) // _T) * _T
    ends = jnp.cumsum(padded)
    starts = ends - padded
    ranks_all = jnp.cumsum(oh32, axis=0) - oh32
    rank = jnp.take_along_axis(ranks_all, plf[:, None], axis=1)[:, 0]
    dest = starts[plf] + rank

    used_blocks = ends[-1] // _T
    bid = jnp.arange(_NB, dtype=jnp.int32)
    src_blk = jnp.minimum(bid, used_blocks - 1)
    blk_expert = jnp.searchsorted(ends, src_blk * _T, side="right").astype(jnp.int32)
    sel = blk_expert

    used = (bid < used_blocks)
    depth_b = jnp.array(_DEPTH, dtype=jnp.int32)[sel]
    lvec = jnp.arange(_MAXD, dtype=jnp.int32)
    is_layer = (used[:, None] & (lvec[None, :] < depth_b[:, None])).astype(jnp.int32)
    is_last = (used[:, None] & (lvec[None, :] == depth_b[:, None] - 1)).astype(jnp.int32)
    rsel_tab = jnp.array(_RSEL_ROWS, dtype=jnp.int32)
    last_tab = jnp.array(_LAST_SLOT, dtype=jnp.int32)
    rsel = jnp.where(used[:, None], rsel_tab[sel], last_tab[sel][:, None])

    x_sorted = jnp.zeros((_NB * _T, _IN), dtype=jnp.float32).at[dest].set(xf)

    w0_bank, b0_bank, wr_bank, br_bank = _pack_weights(params)
    y_sorted = _expert_mlp(x_sorted, sel, rsel, is_layer, is_last,
                           w0_bank, b0_bank, wr_bank, br_bank)

    out = y_sorted[dest]
    return out.reshape(b, n, _OUT)
